# Initial kernel scaffold; baseline (speedup 1.0000x reference)
import jax
import jax.numpy as jnp
from jax.experimental import pallas as pl


def kernel(tc, embedding):
    raise NotImplementedError("write your pallas kernel here")


# speedup vs baseline: 3.4197x; 3.4197x over previous
import functools

import jax
import jax.numpy as jnp
from jax import lax
from jax.experimental import pallas as pl
from jax.experimental.pallas import tpu as pltpu
from jax.experimental.pallas import tpu_sc as plsc

_NC = 2
_NS = 16
_NW = _NC * _NS

_D = 64
_CH = 128


def kernel(tc, embedding):
    n_b, n_h = tc.shape
    total = n_b * n_h
    bpw = total // _NW
    nch = bpw // _CH

    flat_idx = tc.reshape(total)

    mesh = plsc.VectorSubcoreMesh(core_axis_name="c", subcore_axis_name="s")

    @functools.partial(
        pl.kernel,
        out_type=jax.ShapeDtypeStruct((total, _D), jnp.float32),
        mesh=mesh,
        scratch_types=[
            pltpu.VMEM((_CH,), jnp.int32),
            pltpu.VMEM((_CH, _D), jnp.float32),
            pltpu.SemaphoreType.DMA,
        ],
        compiler_params=pltpu.CompilerParams(use_tc_tiling_on_sc=False),
    )
    def run(idx_hbm, table_hbm, out_hbm, idx_v, rows_v, sem):
        wid = lax.axis_index("s") * _NC + lax.axis_index("c")
        base = wid * bpw

        def body(i, carry):
            off = base + i * _CH
            pltpu.sync_copy(idx_hbm.at[pl.ds(off, _CH)], idx_v)
            pltpu.async_copy(table_hbm.at[idx_v], rows_v, sem).wait()
            pltpu.sync_copy(rows_v, out_hbm.at[pl.ds(off, _CH)])
            return carry

        lax.fori_loop(0, nch, body, 0)

    out = run(flat_idx, embedding)
    return out.reshape(n_b, n_h, _D)


# speedup vs baseline: 3.6505x; 1.0675x over previous
import functools

import jax
import jax.numpy as jnp
from jax import lax
from jax.experimental import pallas as pl
from jax.experimental.pallas import tpu as pltpu
from jax.experimental.pallas import tpu_sc as plsc

_NC = 2
_NS = 16
_NW = _NC * _NS

_D = 64
_CH = 128
_NB = 4


def kernel(tc, embedding):
    n_b, n_h = tc.shape
    total = n_b * n_h
    rows_total = total // _CH
    rpw = rows_total // _NW
    ngr = rpw // _NB
    assert rows_total % _NW == 0 and rpw % _NB == 0 and ngr % 2 == 0

    idx2d = tc.reshape(rows_total, _CH)

    mesh = plsc.VectorSubcoreMesh(core_axis_name="c", subcore_axis_name="s")

    @functools.partial(
        pl.kernel,
        out_type=jax.ShapeDtypeStruct((total, _D), jnp.float32),
        mesh=mesh,
        scratch_types=[
            pltpu.VMEM((_NB, _CH), jnp.int32),
            pltpu.VMEM((_NB, _CH), jnp.int32),
            pltpu.VMEM((_NB, _CH, _D), jnp.float32),
            pltpu.VMEM((_NB, _CH, _D), jnp.float32),
            pltpu.SemaphoreType.DMA,
            pltpu.SemaphoreType.DMA,
            pltpu.SemaphoreType.DMA,
            pltpu.SemaphoreType.DMA,
        ],
        compiler_params=pltpu.CompilerParams(use_tc_tiling_on_sc=False),
    )
    def run(idx_hbm, table_hbm, out_hbm,
            idx0, idx1, rows0, rows1, gsem0, gsem1, wsem0, wsem1):
        wid = lax.axis_index("s") * _NC + lax.axis_index("c")
        base_row = wid * rpw

        def gather_group(g, idx_v, rows_v, gsem, wsem):
            grow = base_row + g * _NB
            pltpu.sync_copy(idx_hbm.at[pl.ds(grow, _NB)], idx_v)
            hs = [
                pltpu.async_copy(table_hbm.at[idx_v.at[b]], rows_v.at[b], gsem)
                for b in range(_NB)
            ]
            for h in hs:
                h.wait()
            for b in range(_NB):
                pltpu.async_copy(
                    rows_v.at[b], out_hbm.at[pl.ds((grow + b) * _CH, _CH)], wsem
                )

        def drain_writes(g, rows_v, wsem):
            grow = base_row + g * _NB
            for b in range(_NB):
                pltpu.make_async_copy(
                    rows_v.at[b], out_hbm.at[pl.ds((grow + b) * _CH, _CH)], wsem
                ).wait()

        gather_group(0, idx0, rows0, gsem0, wsem0)
        gather_group(1, idx1, rows1, gsem1, wsem1)

        def body(it, carry):
            g_even = 2 * it + 2
            drain_writes(g_even - 2, rows0, wsem0)
            gather_group(g_even, idx0, rows0, gsem0, wsem0)
            g_odd = g_even + 1
            drain_writes(g_odd - 2, rows1, wsem1)
            gather_group(g_odd, idx1, rows1, gsem1, wsem1)
            return carry

        lax.fori_loop(0, (ngr - 2) // 2, body, 0)

        drain_writes(ngr - 2, rows0, wsem0)
        drain_writes(ngr - 1, rows1, wsem1)

    out = run(idx2d, embedding)
    return out.reshape(n_b, n_h, _D)
